# baseline (speedup 1.0000x reference)
import jax
import jax.numpy as jnp
from jax.experimental import pallas as pl

N = 524288
D = 64
BL = 1 << 20


def _body(x_ref, o_ref):
    o_ref[...] = x_ref[...] * 1.01 + 0.02


def kernel(input, z, scale_table, shift_table):
    xf = input.reshape(-1)
    grid = ((N * D) // BL,)
    out = pl.pallas_call(
        _body,
        grid=grid,
        in_specs=[pl.BlockSpec((BL,), lambda i: (i,))],
        out_specs=pl.BlockSpec((BL,), lambda i: (i,)),
        out_shape=jax.ShapeDtypeStruct((N * D,), jnp.float32),
    )(xf)
    return out.reshape(N, D)


# speedup vs baseline: 1.3531x; 1.3531x over previous
import jax
import jax.numpy as jnp
from jax.experimental import pallas as pl
from jax.experimental.pallas import tpu as pltpu

N = 524288
D = 64
BR = 16384


def _body(x_ref, o_ref):
    o_ref[...] = x_ref[...] * 1.01 + 0.02


def kernel(input, z, scale_table, shift_table):
    grid = (N // BR,)
    return pl.pallas_call(
        _body,
        grid=grid,
        in_specs=[pl.BlockSpec((BR, D), lambda i: (i, 0))],
        out_specs=pl.BlockSpec((BR, D), lambda i: (i, 0)),
        out_shape=jax.ShapeDtypeStruct((N, D), jnp.float32),
        compiler_params=pltpu.CompilerParams(
            dimension_semantics=(pltpu.ARBITRARY,),
        ),
    )(input)


# speedup vs baseline: 1.3558x; 1.0019x over previous
import jax
import jax.numpy as jnp
from jax.experimental import pallas as pl
from jax.experimental.pallas import tpu as pltpu

N = 524288
D = 64
BR = 4096
NBUF = 4
STEPS = N // BR
G = STEPS // NBUF


def _body(x_hbm, o_hbm, xin, xout, sin, sout):
    g = pl.program_id(0)

    def start_in(blk, slot):
        pltpu.make_async_copy(
            x_hbm.at[pl.ds(blk * BR, BR)], xin.at[slot], sin.at[slot]
        ).start()

    def wait_in(slot):
        pltpu.make_async_copy(
            x_hbm.at[pl.ds(0, BR)], xin.at[slot], sin.at[slot]
        ).wait()

    def start_out(blk, slot):
        pltpu.make_async_copy(
            xout.at[slot], o_hbm.at[pl.ds(blk * BR, BR)], sout.at[slot]
        ).start()

    def wait_out(slot):
        pltpu.make_async_copy(
            xout.at[slot], o_hbm.at[pl.ds(0, BR)], sout.at[slot]
        ).wait()

    @pl.when(g == 0)
    def _prime():
        for b in range(NBUF):
            start_in(b, b)

    for b in range(NBUF):
        blk = g * NBUF + b
        wait_in(b)

        @pl.when(g > 0)
        def _drain_out(b=b):
            wait_out(b)

        xout[b] = xin[b] * 1.01 + 0.02

        start_out(blk, b)

        @pl.when(g < G - 1)
        def _next_in(blk=blk, b=b):
            start_in(blk + NBUF, b)

    @pl.when(g == G - 1)
    def _final_drain():
        for b in range(NBUF):
            wait_out(b)


def kernel(input, z, scale_table, shift_table):
    return pl.pallas_call(
        _body,
        grid=(G,),
        in_specs=[pl.BlockSpec(memory_space=pltpu.HBM)],
        out_specs=pl.BlockSpec(memory_space=pltpu.HBM),
        out_shape=jax.ShapeDtypeStruct((N, D), jnp.float32),
        scratch_shapes=[
            pltpu.VMEM((NBUF, BR, D), jnp.float32),
            pltpu.VMEM((NBUF, BR, D), jnp.float32),
            pltpu.SemaphoreType.DMA((NBUF,)),
            pltpu.SemaphoreType.DMA((NBUF,)),
        ],
    )(input)


# speedup vs baseline: 7.1936x; 5.3059x over previous
import jax
import jax.numpy as jnp
from jax import lax
from jax.experimental import pallas as pl

N = 524288
D = 64
BC = 16384


def _body(z_ref, stab_ref, htab_ref, x_ref, o_ref):
    zb = z_ref[...].reshape(1, BC)
    k = lax.broadcasted_iota(jnp.int32, (D, BC), 0)
    e = zb == k
    s = jnp.sum(jnp.where(e, stab_ref[...], 0.0), axis=0, keepdims=True)
    h = jnp.sum(jnp.where(e, htab_ref[...], 0.0), axis=0, keepdims=True)
    o_ref[...] = x_ref[...] * s + h


def kernel(input, z, scale_table, shift_table):
    xt = input.T
    zi = z.astype(jnp.int32)
    stab = jnp.zeros((D, 1), jnp.float32).at[:54, 0].set(scale_table[:, 0])
    htab = jnp.zeros((D, 1), jnp.float32).at[:54, 0].set(shift_table[:, 0])
    grid = (N // BC,)
    out_t = pl.pallas_call(
        _body,
        grid=grid,
        in_specs=[
            pl.BlockSpec((BC,), lambda i: (i,)),
            pl.BlockSpec((D, 1), lambda i: (0, 0)),
            pl.BlockSpec((D, 1), lambda i: (0, 0)),
            pl.BlockSpec((D, BC), lambda i: (0, i)),
        ],
        out_specs=pl.BlockSpec((D, BC), lambda i: (0, i)),
        out_shape=jax.ShapeDtypeStruct((D, N), jnp.float32),
    )(zi, stab, htab, xt)
    return out_t.T


# speedup vs baseline: 7.5646x; 1.0516x over previous
import jax
import jax.numpy as jnp
from jax import lax
from jax.experimental import pallas as pl

N = 524288
D = 64
BC = 32768


def _body(z_ref, stab_ref, htab_ref, x_ref, o_ref):
    zb = z_ref[...].reshape(1, BC)
    k = lax.broadcasted_iota(jnp.int32, (D, BC), 0)
    e = zb == k
    s = jnp.sum(jnp.where(e, stab_ref[...], 0.0), axis=0, keepdims=True)
    h = jnp.sum(jnp.where(e, htab_ref[...], 0.0), axis=0, keepdims=True)
    o_ref[...] = x_ref[...] * s + h


def kernel(input, z, scale_table, shift_table):
    xt = input.T
    zi = z.astype(jnp.int32)
    stab = jnp.zeros((D, 1), jnp.float32).at[:54, 0].set(scale_table[:, 0])
    htab = jnp.zeros((D, 1), jnp.float32).at[:54, 0].set(shift_table[:, 0])
    grid = (N // BC,)
    out_t = pl.pallas_call(
        _body,
        grid=grid,
        in_specs=[
            pl.BlockSpec((BC,), lambda i: (i,)),
            pl.BlockSpec((D, 1), lambda i: (0, 0)),
            pl.BlockSpec((D, 1), lambda i: (0, 0)),
            pl.BlockSpec((D, BC), lambda i: (0, i)),
        ],
        out_specs=pl.BlockSpec((D, BC), lambda i: (0, i)),
        out_shape=jax.ShapeDtypeStruct((D, N), jnp.float32),
    )(zi, stab, htab, xt)
    return out_t.T
